# baseline (speedup 1.0000x reference)
import functools

import jax
import jax.numpy as jnp
from jax import lax
from jax.experimental import pallas as pl
from jax.experimental.pallas import tpu as pltpu
from jax.experimental.pallas import tpu_sc as plsc

DELAY = 2048
NUM_ENVS = 16384

_NC = 2
_NS = 16
_NW = _NC * _NS
_BPW = NUM_ENVS // _NW
_W = 128
_NB = DELAY // _W
_L = 16
_NCH = _BPW // _L
_SLOTS = 768
_NDMA = _SLOTS // _L

_IN_BOUNDS = "wrap"


def _gather_body(i_hbm, buf_hbm, out_hbm, iv_v, idxc_v, pos_v, vals_v, ans_v,
                 sem):
    wid = lax.axis_index("s") * _NC + lax.axis_index("c")
    base = wid * _BPW
    lane = lax.iota(jnp.int32, _L)
    zero16 = lane * 0

    pltpu.sync_copy(i_hbm.at[pl.ds(base, _BPW)], iv_v)

    def init_body(t, carry):
        idxc_v[pl.ds(t * _L, _L)] = zero16
        return carry

    lax.fori_loop(0, _NDMA, init_body, 0)

    def hist_body(t, cnt16):
        iv16 = iv_v[pl.ds(t * _L, _L)]
        cb16 = lax.shift_right_logical(iv16, 7)
        for w in range(_NB):
            n_w = jnp.sum((cb16 == w).astype(jnp.int32))
            cnt16 = jnp.where(lane == w, cnt16 + n_w, cnt16)
        return cnt16

    cnt16 = lax.fori_loop(0, _NCH, hist_body, zero16)

    a16 = jnp.bitwise_and(cnt16 + (_L - 1), -_L)
    seg16 = plsc.cumsum(a16) - a16
    ends16 = seg16 + a16

    def scat_body(t, run16):
        iv16 = iv_v[pl.ds(t * _L, _L)]
        cb16 = lax.shift_right_logical(iv16, 7)
        rank16 = zero16
        newrun = run16
        for w in range(_NB):
            m = cb16 == w
            mi = m.astype(jnp.int32)
            csum = plsc.cumsum(mi)
            rank16 = jnp.where(m, csum - 1, rank16)
            n_w = jnp.sum(mi)
            newrun = jnp.where(lane == w, newrun + n_w, newrun)
        segg = jnp.take(seg16, cb16, mode=_IN_BOUNDS)
        rung = jnp.take(run16, cb16, mode=_IN_BOUNDS)
        pos16 = segg + rung + rank16
        plsc.store_scatter(idxc_v, [pos16], base + t * _L + lane)
        pos_v[pl.ds(t * _L, _L)] = pos16
        return newrun

    lax.fori_loop(0, _NCH, scat_body, zero16)

    end_scalars = [jnp.sum(jnp.where(lane == w, ends16, 0)) for w in range(_NB)]
    chunk_w = []
    for j in range(_NDMA // _L):
        slot16 = (lane + _L * j) * _L
        w_j = zero16
        for w in range(_NB):
            w_j = jnp.where(slot16 >= end_scalars[w], w_j + 1, w_j)
        chunk_w.append(jnp.minimum(w_j, _NB - 1))

    cps = []
    for c in range(_NDMA):
        j, lpos = divmod(c, _L)
        wc = jnp.max(jnp.where(lane == lpos, chunk_w[j], 0))
        src = buf_hbm.at[
            plsc.Indices(idxc_v.at[pl.ds(c * _L, _L)]),
            pl.ds(wc * _W, _W),
        ]
        cps.append(pltpu.async_copy(src, vals_v.at[pl.ds(c * _L, _L)], sem))
    for cp in cps:
        cp.wait()

    def ext_body(t, carry):
        sl = pl.ds(t * _L, _L)
        iv16 = iv_v[sl]
        col16 = iv16 & (_W - 1)
        pos16 = pos_v[sl]
        ans_v[sl] = plsc.load_gather(vals_v, [pos16, col16])
        return carry

    lax.fori_loop(0, _NCH, ext_body, 0)

    pltpu.sync_copy(ans_v, out_hbm.at[pl.ds(base, _BPW)])


@functools.partial(
    pl.kernel,
    mesh=plsc.VectorSubcoreMesh(core_axis_name="c", subcore_axis_name="s"),
    out_type=jax.ShapeDtypeStruct((NUM_ENVS,), jnp.float32),
    compiler_params=pltpu.CompilerParams(needs_layout_passes=False),
    scratch_types=[
        pltpu.VMEM((_BPW,), jnp.int32),
        pltpu.VMEM((_SLOTS,), jnp.int32),
        pltpu.VMEM((_BPW,), jnp.int32),
        pltpu.VMEM((_SLOTS, _W), jnp.float32),
        pltpu.VMEM((_BPW,), jnp.float32),
        pltpu.SemaphoreType.DMA,
    ],
)
def _sc_gather(i_hbm, buf_hbm, out_hbm, iv_v, idxc_v, pos_v, vals_v, ans_v,
               sem):
    _gather_body(i_hbm, buf_hbm, out_hbm, iv_v, idxc_v, pos_v, vals_v, ans_v,
                 sem)


def kernel(x, buffor, i):
    del x
    return _sc_gather(i, buffor)


# speedup vs baseline: 9.0379x; 9.0379x over previous
import functools

import jax
import jax.numpy as jnp
from jax import lax
from jax.experimental import pallas as pl
from jax.experimental.pallas import tpu as pltpu
from jax.experimental.pallas import tpu_sc as plsc

DELAY = 2048
NUM_ENVS = 16384

_NC = 2
_NS = 16
_NW = _NC * _NS
_BPW = NUM_ENVS // _NW
_W = 128
_L = 16
_NCH = _BPW // _L
_CH = 128
_ND = _BPW // _CH
_RUNS_PER_RBLK = (DELAY // _W) * 8


def _gather_body(i_hbm, buf_hbm, out_hbm, iv_v, idx0, idx1, idx2, idx3,
                 vals_v, ans_v, sem):
    idxc = (idx0, idx1, idx2, idx3)
    wid = lax.axis_index("s") * _NC + lax.axis_index("c")
    base = wid * _BPW
    lane = lax.iota(jnp.int32, _L)

    pltpu.sync_copy(i_hbm.at[pl.ds(base, _BPW)], iv_v)

    for c in range(_ND):
        for k in range(_CH // _L):
            t = c * (_CH // _L) + k
            sl = pl.ds(t * _L, _L)
            iv16 = iv_v[sl]
            r16 = base + t * _L + lane
            j16 = (
                lax.shift_left(lax.shift_right_logical(r16, 3), 7)
                + lax.shift_left(lax.shift_right_logical(iv16, 7), 3)
                + (r16 & 7)
            )
            idxc[c][pl.ds(k * _L, _L)] = j16

    cps = [
        pltpu.async_copy(
            buf_hbm.at[plsc.Indices(idxc[c])],
            vals_v.at[pl.ds(c * _CH, _CH)],
            sem,
        )
        for c in range(_ND)
    ]
    for cp in cps:
        cp.wait()

    def ext_body(t, carry):
        sl = pl.ds(t * _L, _L)
        iv16 = iv_v[sl]
        col16 = iv16 & (_W - 1)
        epos16 = lane + t * _L
        ans_v[sl] = plsc.load_gather(vals_v, [epos16, col16])
        return carry

    lax.fori_loop(0, _NCH, ext_body, 0)

    pltpu.sync_copy(ans_v, out_hbm.at[pl.ds(base, _BPW)])


@functools.partial(
    pl.kernel,
    mesh=plsc.VectorSubcoreMesh(core_axis_name="c", subcore_axis_name="s"),
    out_type=jax.ShapeDtypeStruct((NUM_ENVS,), jnp.float32),
    compiler_params=pltpu.CompilerParams(needs_layout_passes=False),
    scratch_types=[
        pltpu.VMEM((_BPW,), jnp.int32),
        pltpu.VMEM((_CH,), jnp.int32),
        pltpu.VMEM((_CH,), jnp.int32),
        pltpu.VMEM((_CH,), jnp.int32),
        pltpu.VMEM((_CH,), jnp.int32),
        pltpu.VMEM((_BPW, _W), jnp.float32),
        pltpu.VMEM((_BPW,), jnp.float32),
        pltpu.SemaphoreType.DMA,
    ],
)
def _sc_gather(i_hbm, buf_hbm, out_hbm, iv_v, idx0, idx1, idx2, idx3,
               vals_v, ans_v, sem):
    _gather_body(i_hbm, buf_hbm, out_hbm, iv_v, idx0, idx1, idx2, idx3,
                 vals_v, ans_v, sem)


def kernel(x, buffor, i):
    del x
    runs = buffor.reshape(NUM_ENVS // 8, 8, DELAY // _W, _W)
    runs = runs.transpose(0, 2, 1, 3)
    runs = runs.reshape(NUM_ENVS // 8 * (DELAY // _W) * 8, _W)
    return _sc_gather(i, runs)


# speedup vs baseline: 10.0395x; 1.1108x over previous
import functools

import jax
import jax.numpy as jnp
from jax import lax
from jax.experimental import pallas as pl
from jax.experimental.pallas import tpu as pltpu
from jax.experimental.pallas import tpu_sc as plsc

DELAY = 2048
NUM_ENVS = 16384

_NC = 2
_NS = 16
_NW = _NC * _NS
_BPW = NUM_ENVS // _NW
_L = 16
_CH = 128
_ND = _BPW // _CH


def _gather_body(i_hbm, buf_hbm, out_hbm, iv_v, idx0, idx1, idx2, idx3,
                 vals_v, sem):
    idxc = (idx0, idx1, idx2, idx3)
    wid = lax.axis_index("s") * _NC + lax.axis_index("c")
    base = wid * _BPW
    lane = lax.iota(jnp.int32, _L)

    pltpu.sync_copy(i_hbm.at[pl.ds(base, _BPW)], iv_v)

    for c in range(_ND):
        for k in range(_CH // _L):
            t = c * (_CH // _L) + k
            sl = pl.ds(t * _L, _L)
            iv16 = iv_v[sl]
            r16 = base + t * _L + lane
            w16 = (
                lax.shift_left(lax.shift_right_logical(r16, 3), 14)
                + lax.shift_left(lax.shift_right_logical(iv16, 7), 10)
                + lax.shift_left(r16 & 7, 7)
                + (iv16 & 127)
            )
            idxc[c][pl.ds(k * _L, _L)] = w16

    cps = [
        pltpu.async_copy(
            buf_hbm.at[plsc.Indices(idxc[c])],
            vals_v.at[pl.ds(c * _CH, _CH)],
            sem,
        )
        for c in range(_ND)
    ]
    for cp in cps:
        cp.wait()

    pltpu.sync_copy(vals_v, out_hbm.at[pl.ds(base, _BPW)])


@functools.partial(
    pl.kernel,
    mesh=plsc.VectorSubcoreMesh(core_axis_name="c", subcore_axis_name="s"),
    out_type=jax.ShapeDtypeStruct((NUM_ENVS,), jnp.float32),
    scratch_types=[
        pltpu.VMEM((_BPW,), jnp.int32),
        pltpu.VMEM((_CH,), jnp.int32),
        pltpu.VMEM((_CH,), jnp.int32),
        pltpu.VMEM((_CH,), jnp.int32),
        pltpu.VMEM((_CH,), jnp.int32),
        pltpu.VMEM((_BPW,), jnp.float32),
        pltpu.SemaphoreType.DMA,
    ],
)
def _sc_gather(i_hbm, buf_hbm, out_hbm, iv_v, idx0, idx1, idx2, idx3,
               vals_v, sem):
    _gather_body(i_hbm, buf_hbm, out_hbm, iv_v, idx0, idx1, idx2, idx3,
                 vals_v, sem)


def kernel(x, buffor, i):
    del x
    flat = buffor.reshape(NUM_ENVS // 8, 8, DELAY // 128, 128)
    flat = flat.transpose(0, 2, 1, 3)
    flat = flat.reshape(NUM_ENVS * DELAY)
    return _sc_gather(i, flat)


# speedup vs baseline: 10.1869x; 1.0147x over previous
import functools

import jax
import jax.numpy as jnp
from jax import lax
from jax.experimental import pallas as pl
from jax.experimental.pallas import tpu as pltpu
from jax.experimental.pallas import tpu_sc as plsc

DELAY = 2048
NUM_ENVS = 16384

_NC = 1
_NS = 16
_NW = _NC * _NS
_BPW = NUM_ENVS // _NW
_L = 16
_CH = 128
_ND = _BPW // _CH


def _gather_body(i_hbm, buf_hbm, out_hbm, iv_v, *rest):
    idxc, (vals_v, sem) = rest[:_ND], rest[_ND:]
    wid = lax.axis_index("s") * _NC + lax.axis_index("c")
    base = wid * _BPW
    lane = lax.iota(jnp.int32, _L)

    pltpu.sync_copy(i_hbm.at[pl.ds(base, _BPW)], iv_v)

    for c in range(_ND):
        for k in range(_CH // _L):
            t = c * (_CH // _L) + k
            sl = pl.ds(t * _L, _L)
            iv16 = iv_v[sl]
            r16 = base + t * _L + lane
            w16 = (
                lax.shift_left(lax.shift_right_logical(r16, 3), 14)
                + lax.shift_left(lax.shift_right_logical(iv16, 7), 10)
                + lax.shift_left(r16 & 7, 7)
                + (iv16 & 127)
            )
            idxc[c][pl.ds(k * _L, _L)] = w16

    cps = [
        pltpu.async_copy(
            buf_hbm.at[plsc.Indices(idxc[c])],
            vals_v.at[pl.ds(c * _CH, _CH)],
            sem,
        )
        for c in range(_ND)
    ]
    for cp in cps:
        cp.wait()

    pltpu.sync_copy(vals_v, out_hbm.at[pl.ds(base, _BPW)])


@functools.partial(
    pl.kernel,
    mesh=plsc.VectorSubcoreMesh(core_axis_name="c", subcore_axis_name="s", num_cores=1),
    out_type=jax.ShapeDtypeStruct((NUM_ENVS,), jnp.float32),
    scratch_types=[
        pltpu.VMEM((_BPW,), jnp.int32),
        *[pltpu.VMEM((_CH,), jnp.int32) for _ in range(_ND)],
        pltpu.VMEM((_BPW,), jnp.float32),
        pltpu.SemaphoreType.DMA,
    ],
)
def _sc_gather(i_hbm, buf_hbm, out_hbm, iv_v, *rest):
    _gather_body(i_hbm, buf_hbm, out_hbm, iv_v, *rest)


def kernel(x, buffor, i):
    del x
    flat = buffor.reshape(NUM_ENVS // 8, 8, DELAY // 128, 128)
    flat = flat.transpose(0, 2, 1, 3)
    flat = flat.reshape(NUM_ENVS * DELAY)
    return _sc_gather(i, flat)


# speedup vs baseline: 10.3366x; 1.0147x over previous
import functools

import jax
import jax.numpy as jnp
from jax import lax
from jax.experimental import pallas as pl
from jax.experimental.pallas import tpu as pltpu
from jax.experimental.pallas import tpu_sc as plsc

DELAY = 2048
NUM_ENVS = 16384

_NC = 1
_NS = 16
_NW = _NC * _NS
_BPW = NUM_ENVS // _NW
_L = 16
_CH = 128
_ND = _BPW // _CH


def _gather_body(i_hbm, buf_hbm, out_hbm, iv_v, *rest):
    idxc, (vals_v, sem) = rest[:_ND], rest[_ND:]
    wid = lax.axis_index("s") * _NC + lax.axis_index("c")
    base = wid * _BPW
    lane = lax.iota(jnp.int32, _L)

    pltpu.sync_copy(i_hbm.at[pl.ds(base, _BPW)], iv_v)

    cps = []
    for c in range(_ND):
        for k in range(_CH // _L):
            t = c * (_CH // _L) + k
            sl = pl.ds(t * _L, _L)
            iv16 = iv_v[sl]
            r16 = base + t * _L + lane
            w16 = (
                lax.shift_left(lax.shift_right_logical(r16, 3), 14)
                + lax.shift_left(lax.shift_right_logical(iv16, 7), 10)
                + lax.shift_left(r16 & 7, 7)
                + (iv16 & 127)
            )
            idxc[c][pl.ds(k * _L, _L)] = w16
        cps.append(
            pltpu.async_copy(
                buf_hbm.at[plsc.Indices(idxc[c])],
                vals_v.at[pl.ds(c * _CH, _CH)],
                sem,
            )
        )
    for cp in cps:
        cp.wait()

    pltpu.sync_copy(vals_v, out_hbm.at[pl.ds(base, _BPW)])


@functools.partial(
    pl.kernel,
    mesh=plsc.VectorSubcoreMesh(core_axis_name="c", subcore_axis_name="s", num_cores=1),
    out_type=jax.ShapeDtypeStruct((NUM_ENVS,), jnp.float32),
    scratch_types=[
        pltpu.VMEM((_BPW,), jnp.int32),
        *[pltpu.VMEM((_CH,), jnp.int32) for _ in range(_ND)],
        pltpu.VMEM((_BPW,), jnp.float32),
        pltpu.SemaphoreType.DMA,
    ],
)
def _sc_gather(i_hbm, buf_hbm, out_hbm, iv_v, *rest):
    _gather_body(i_hbm, buf_hbm, out_hbm, iv_v, *rest)


def kernel(x, buffor, i):
    del x
    flat = buffor.reshape(NUM_ENVS // 8, 8, DELAY // 128, 128)
    flat = flat.transpose(0, 2, 1, 3)
    flat = flat.reshape(NUM_ENVS * DELAY)
    return _sc_gather(i, flat)
